# Initial kernel scaffold; baseline (speedup 1.0000x reference)
import jax
import jax.numpy as jnp
from jax.experimental import pallas as pl


def kernel(x_cat, x_num, emb, W1, b1, W2, b2, W3, b3, W4, b4):
    raise NotImplementedError("write your pallas kernel here")


# speedup vs baseline: 2.7230x; 2.7230x over previous
import jax
import jax.numpy as jnp
from jax import lax
from jax.experimental import pallas as pl
from jax.experimental.pallas import tpu as pltpu
from jax.experimental.pallas import tpu_sc as plsc

_NUM_FIELDS = 26
_VOCAB = 1000
_EMB_DIM = 50
_EMB_PAD = 128
_BATCH = 4096
_NW = 32
_ROWS_PER_W = _BATCH * _NUM_FIELDS // _NW
_G = 128
_NG = _ROWS_PER_W // _G
_SPC = 2
_CH = _SPC * _G
_NCH = _ROWS_PER_W // _CH


def _sc_gather_body(xcat_hbm, table_hbm, out_hbm, xi_v, idx_v, rows_v,
                    sem, sem2):
    nc = 2
    wid = lax.axis_index("s") * nc + lax.axis_index("c")
    base = wid * _ROWS_PER_W

    pltpu.sync_copy(xcat_hbm.at[pl.ds(base, _ROWS_PER_W)], xi_v)

    def build_row(g, _):
        for j in range(_G // 16):
            off = pl.multiple_of(g * _G + j * 16, 16)
            p = base + off + lax.iota(jnp.int32, 16)
            v = xi_v[pl.ds(off, 16)] + (p % _NUM_FIELDS) * _VOCAB
            idx_v[g, pl.ds(j * 16, 16)] = v
        return 0

    lax.fori_loop(0, _NG, build_row, 0, unroll=False)

    store_cps = [None, None]
    for c in range(_NCH):
        b = c % 2
        if store_cps[b] is not None:
            store_cps[b].wait()
        gcps = []
        for s in range(_SPC):
            g = c * _SPC + s
            gcps.append(pltpu.async_copy(
                table_hbm.at[idx_v.at[g]],
                rows_v.at[b, pl.ds(s * _G, _G)],
                sem,
            ))
        for cp in gcps:
            cp.wait()
        store_cps[b] = pltpu.async_copy(
            rows_v.at[b],
            out_hbm.at[pl.ds(base + c * _CH, _CH)],
            sem2,
        )
    for cp in store_cps:
        if cp is not None:
            cp.wait()


def _sc_gather(xcat_flat, table_pad):
    mesh = plsc.VectorSubcoreMesh(core_axis_name="c", subcore_axis_name="s")
    k = pl.kernel(
        _sc_gather_body,
        out_type=jax.ShapeDtypeStruct((_BATCH * _NUM_FIELDS, _EMB_PAD),
                                      jnp.float32),
        mesh=mesh,
        scratch_types=[
            pltpu.VMEM((_ROWS_PER_W,), jnp.int32),
            pltpu.VMEM((_NG, _G), jnp.int32),
            pltpu.VMEM((2, _CH, _EMB_PAD), jnp.float32),
            pltpu.SemaphoreType.DMA,
            pltpu.SemaphoreType.DMA,
        ],
    )
    return k(xcat_flat, table_pad)


def _mlp_body(xc_ref, xn_ref, w1a_ref, w1b_ref, b1_ref, w2_ref, b2_ref,
              w3_ref, b3_ref, w4_ref, b4_ref, out_ref):
    f32 = jnp.float32
    h = jnp.dot(xc_ref[...], w1a_ref[...], preferred_element_type=f32)
    h += jnp.dot(xn_ref[...], w1b_ref[...], preferred_element_type=f32)
    h = jnp.maximum(h + b1_ref[...], 0.0)
    e = jnp.dot(h, w2_ref[...], preferred_element_type=f32) + b2_ref[...]
    h2 = jnp.dot(e, w3_ref[...], preferred_element_type=f32) + b3_ref[...]
    h2 = jnp.maximum(h2, 0.0)
    out_ref[...] = (jnp.dot(h2, w4_ref[...], preferred_element_type=f32)
                    + b4_ref[...])


def _mlp(xc, x_num, w1a, w1b, b1, w2, b2, w3, b3, w4, b4, bm=512):
    nb = _BATCH // bm
    full = lambda shape: pl.BlockSpec(shape, lambda i: (0, 0))
    return pl.pallas_call(
        _mlp_body,
        grid=(nb,),
        in_specs=[
            pl.BlockSpec((bm, xc.shape[1]), lambda i: (i, 0)),
            pl.BlockSpec((bm, x_num.shape[1]), lambda i: (i, 0)),
            full(w1a.shape), full(w1b.shape), full(b1.shape),
            full(w2.shape), full(b2.shape),
            full(w3.shape), full(b3.shape),
            full(w4.shape), full(b4.shape),
        ],
        out_specs=pl.BlockSpec((bm, w4.shape[1]), lambda i: (i, 0)),
        out_shape=jax.ShapeDtypeStruct((_BATCH, w4.shape[1]), jnp.float32),
    )(xc, x_num, w1a, w1b, b1, w2, b2, w3, b3, w4, b4)


@jax.jit
def kernel(x_cat, x_num, emb, W1, b1, W2, b2, W3, b3, W4, b4):
    table_pad = jnp.pad(
        emb.reshape(_NUM_FIELDS * _VOCAB, _EMB_DIM),
        ((0, 0), (0, _EMB_PAD - _EMB_DIM)))
    xcat_flat = x_cat.reshape(-1)
    xcp = _sc_gather(xcat_flat, table_pad)
    xcp = xcp.reshape(_BATCH, _NUM_FIELDS * _EMB_PAD)
    w1a = W1[:_NUM_FIELDS * _EMB_DIM]
    w1a_pad = (jnp.zeros((_NUM_FIELDS, _EMB_PAD, 1024), jnp.float32)
               .at[:, :_EMB_DIM, :]
               .set(w1a.reshape(_NUM_FIELDS, _EMB_DIM, 1024))
               .reshape(_NUM_FIELDS * _EMB_PAD, 1024))
    w1b = W1[_NUM_FIELDS * _EMB_DIM:]
    return _mlp(xcp, x_num, w1a_pad, w1b, b1.reshape(1, -1), W2,
                b2.reshape(1, -1), W3, b3.reshape(1, -1), W4,
                b4.reshape(1, -1))


# speedup vs baseline: 2.8085x; 1.0314x over previous
import jax
import jax.numpy as jnp
from jax import lax
from jax.experimental import pallas as pl
from jax.experimental.pallas import tpu as pltpu
from jax.experimental.pallas import tpu_sc as plsc

_NUM_FIELDS = 26
_VOCAB = 1000
_EMB_DIM = 50
_EMB_PAD = 128
_BATCH = 4096
_NW = 32
_ROWS_PER_W = _BATCH * _NUM_FIELDS // _NW
_G = 128
_NG = _ROWS_PER_W // _G
_SPC = 2
_CH = _SPC * _G
_NCH = _ROWS_PER_W // _CH


def _sc_gather_body(xcat_hbm, table_hbm, out_hbm, xi_v, idx_v, rows_v,
                    sem, sem2):
    nc = 2
    wid = lax.axis_index("s") * nc + lax.axis_index("c")
    base = wid * _ROWS_PER_W

    pltpu.sync_copy(xcat_hbm.at[pl.ds(base, _ROWS_PER_W)], xi_v)

    def build_row(g, _):
        for j in range(_G // 16):
            off = pl.multiple_of(g * _G + j * 16, 16)
            p = base + off + lax.iota(jnp.int32, 16)
            v = xi_v[pl.ds(off, 16)] + (p % _NUM_FIELDS) * _VOCAB
            idx_v[g, pl.ds(j * 16, 16)] = v
        return 0

    lax.fori_loop(0, _NG, build_row, 0, unroll=False)

    store_cps = [None, None]
    for c in range(_NCH):
        b = c % 2
        if store_cps[b] is not None:
            store_cps[b].wait()
        gcps = []
        for s in range(_SPC):
            g = c * _SPC + s
            gcps.append(pltpu.async_copy(
                table_hbm.at[idx_v.at[g]],
                rows_v.at[b, pl.ds(s * _G, _G)],
                sem,
            ))
        for cp in gcps:
            cp.wait()
        store_cps[b] = pltpu.async_copy(
            rows_v.at[b],
            out_hbm.at[pl.ds(base + c * _CH, _CH)],
            sem2,
        )
    for cp in store_cps:
        if cp is not None:
            cp.wait()


def _sc_gather(xcat_flat, table_pad):
    mesh = plsc.VectorSubcoreMesh(core_axis_name="c", subcore_axis_name="s")
    k = pl.kernel(
        _sc_gather_body,
        out_type=jax.ShapeDtypeStruct((_BATCH * _NUM_FIELDS, _EMB_PAD),
                                      jnp.float32),
        mesh=mesh,
        scratch_types=[
            pltpu.VMEM((_ROWS_PER_W,), jnp.int32),
            pltpu.VMEM((_NG, _G), jnp.int32),
            pltpu.VMEM((2, _CH, _EMB_PAD), jnp.float32),
            pltpu.SemaphoreType.DMA,
            pltpu.SemaphoreType.DMA,
        ],
    )
    return k(xcat_flat, table_pad)


def _mlp_body(xc_ref, xn_ref, w1a_ref, w1b_ref, b1_ref, w2_ref, b2_ref,
              w3_ref, b3_ref, w4_ref, b4_ref, out_ref):
    f32, bf16 = jnp.float32, jnp.bfloat16
    h = jnp.dot(xc_ref[...].astype(bf16), w1a_ref[...],
                preferred_element_type=f32)
    h += jnp.dot(xn_ref[...].astype(bf16), w1b_ref[...],
                 preferred_element_type=f32)
    h = jnp.maximum(h + b1_ref[...], 0.0)
    e = jnp.dot(h.astype(bf16), w2_ref[...],
                preferred_element_type=f32) + b2_ref[...]
    h2 = jnp.dot(e.astype(bf16), w3_ref[...],
                 preferred_element_type=f32) + b3_ref[...]
    h2 = jnp.maximum(h2, 0.0)
    out_ref[...] = (jnp.dot(h2.astype(bf16), w4_ref[...],
                            preferred_element_type=f32) + b4_ref[...])


def _mlp(xc, x_num, w1a, w1b, b1, w2, b2, w3, b3, w4, b4, bm=512):
    nb = _BATCH // bm
    full = lambda shape: pl.BlockSpec(shape, lambda i: (0, 0))
    return pl.pallas_call(
        _mlp_body,
        grid=(nb,),
        in_specs=[
            pl.BlockSpec((bm, xc.shape[1]), lambda i: (i, 0)),
            pl.BlockSpec((bm, x_num.shape[1]), lambda i: (i, 0)),
            full(w1a.shape), full(w1b.shape), full(b1.shape),
            full(w2.shape), full(b2.shape),
            full(w3.shape), full(b3.shape),
            full(w4.shape), full(b4.shape),
        ],
        out_specs=pl.BlockSpec((bm, w4.shape[1]), lambda i: (i, 0)),
        out_shape=jax.ShapeDtypeStruct((_BATCH, w4.shape[1]), jnp.float32),
    )(xc, x_num, w1a, w1b, b1, w2, b2, w3, b3, w4, b4)


@jax.jit
def kernel(x_cat, x_num, emb, W1, b1, W2, b2, W3, b3, W4, b4):
    table_pad = jnp.pad(
        emb.reshape(_NUM_FIELDS * _VOCAB, _EMB_DIM),
        ((0, 0), (0, _EMB_PAD - _EMB_DIM)))
    xcat_flat = x_cat.reshape(-1)
    xcp = _sc_gather(xcat_flat, table_pad)
    xcp = xcp.reshape(_BATCH, _NUM_FIELDS * _EMB_PAD)
    bf16 = jnp.bfloat16
    w1a = W1[:_NUM_FIELDS * _EMB_DIM].astype(bf16)
    w1a_pad = (jnp.zeros((_NUM_FIELDS, _EMB_PAD, 1024), bf16)
               .at[:, :_EMB_DIM, :]
               .set(w1a.reshape(_NUM_FIELDS, _EMB_DIM, 1024))
               .reshape(_NUM_FIELDS * _EMB_PAD, 1024))
    w1b = W1[_NUM_FIELDS * _EMB_DIM:].astype(bf16)
    return _mlp(xcp, x_num, w1a_pad, w1b, b1.reshape(1, -1),
                W2.astype(bf16), b2.reshape(1, -1), W3.astype(bf16),
                b3.reshape(1, -1), W4.astype(bf16), b4.reshape(1, -1))


# speedup vs baseline: 3.1879x; 1.1351x over previous
import jax
import jax.numpy as jnp
from jax import lax
from jax.experimental import pallas as pl
from jax.experimental.pallas import tpu as pltpu
from jax.experimental.pallas import tpu_sc as plsc

_NUM_FIELDS = 26
_VOCAB = 1000
_EMB_DIM = 50
_EMB_PAD = 128
_BATCH = 4096
_NW = 32
_BB = 256
_NBLK = _BATCH // _BB
_UNITS_PER_W = _NUM_FIELDS * _NBLK // _NW
_G = 128


def _sc_gather_body(xcatT_hbm, table_hbm, out_hbm, idx_v, rows_v, sem, sem2):
    wid = lax.axis_index("s") * 2 + lax.axis_index("c")
    store_cps = [None, None]
    for j in range(_UNITS_PER_W):
        u = wid * _UNITS_PER_W + j
        f = u // _NBLK
        boff = pl.multiple_of((u % _NBLK) * _BB, _BB)
        b = j % 2
        pltpu.sync_copy(xcatT_hbm.at[f, pl.ds(boff, _BB)], idx_v.at[b])
        for k in range(_BB // 16):
            s = pl.ds(k * 16, 16)
            idx_v[b, s] = idx_v[b, s] + f * _VOCAB
        if store_cps[b] is not None:
            store_cps[b].wait()
        gcps = []
        for s in range(_BB // _G):
            gcps.append(pltpu.async_copy(
                table_hbm.at[idx_v.at[b, pl.ds(s * _G, _G)]],
                rows_v.at[b, pl.ds(s * _G, _G)],
                sem,
            ))
        for cp in gcps:
            cp.wait()
        store_cps[b] = pltpu.async_copy(
            rows_v.at[b],
            out_hbm.at[f, pl.ds(boff, _BB)],
            sem2,
        )
    for cp in store_cps:
        if cp is not None:
            cp.wait()


def _sc_gather(xcat_T, table_pad):
    mesh = plsc.VectorSubcoreMesh(core_axis_name="c", subcore_axis_name="s")
    k = pl.kernel(
        _sc_gather_body,
        out_type=jax.ShapeDtypeStruct((_NUM_FIELDS, _BATCH, _EMB_PAD),
                                      jnp.float32),
        mesh=mesh,
        scratch_types=[
            pltpu.VMEM((2, _BB), jnp.int32),
            pltpu.VMEM((2, _BB, _EMB_PAD), jnp.float32),
            pltpu.SemaphoreType.DMA,
            pltpu.SemaphoreType.DMA,
        ],
    )
    return k(xcat_T, table_pad)


def _mlp_body(xc_ref, xn_ref, w1p_ref, w1b_ref, b1_ref, w2_ref, b2_ref,
              w3_ref, b3_ref, w4_ref, b4_ref, out_ref):
    f32, bf16 = jnp.float32, jnp.bfloat16
    h = jnp.dot(xn_ref[...].astype(bf16), w1b_ref[...],
                preferred_element_type=f32)
    for f in range(_NUM_FIELDS):
        h += jnp.dot(xc_ref[f].astype(bf16), w1p_ref[f],
                     preferred_element_type=f32)
    h = jnp.maximum(h + b1_ref[...], 0.0)
    e = jnp.dot(h.astype(bf16), w2_ref[...],
                preferred_element_type=f32) + b2_ref[...]
    h2 = jnp.dot(e.astype(bf16), w3_ref[...],
                 preferred_element_type=f32) + b3_ref[...]
    h2 = jnp.maximum(h2, 0.0)
    out_ref[...] = (jnp.dot(h2.astype(bf16), w4_ref[...],
                            preferred_element_type=f32) + b4_ref[...])


def _mlp(xc3, x_num, w1p, w1b, b1, w2, b2, w3, b3, w4, b4, bm=512):
    nb = _BATCH // bm
    full2 = lambda shape: pl.BlockSpec(shape, lambda i: (0, 0))
    return pl.pallas_call(
        _mlp_body,
        grid=(nb,),
        in_specs=[
            pl.BlockSpec((_NUM_FIELDS, bm, _EMB_PAD), lambda i: (0, i, 0)),
            pl.BlockSpec((bm, x_num.shape[1]), lambda i: (i, 0)),
            pl.BlockSpec(w1p.shape, lambda i: (0, 0, 0)),
            full2(w1b.shape), full2(b1.shape),
            full2(w2.shape), full2(b2.shape),
            full2(w3.shape), full2(b3.shape),
            full2(w4.shape), full2(b4.shape),
        ],
        out_specs=pl.BlockSpec((bm, w4.shape[1]), lambda i: (i, 0)),
        out_shape=jax.ShapeDtypeStruct((_BATCH, w4.shape[1]), jnp.float32),
    )(xc3, x_num, w1p, w1b, b1, w2, b2, w3, b3, w4, b4)


@jax.jit
def kernel(x_cat, x_num, emb, W1, b1, W2, b2, W3, b3, W4, b4):
    table_pad = jnp.pad(
        emb.reshape(_NUM_FIELDS * _VOCAB, _EMB_DIM),
        ((0, 0), (0, _EMB_PAD - _EMB_DIM)))
    xc3 = _sc_gather(x_cat.T, table_pad)
    bf16 = jnp.bfloat16
    w1a = W1[:_NUM_FIELDS * _EMB_DIM].astype(bf16)
    w1p = (jnp.zeros((_NUM_FIELDS, _EMB_PAD, 1024), bf16)
           .at[:, :_EMB_DIM, :]
           .set(w1a.reshape(_NUM_FIELDS, _EMB_DIM, 1024)))
    w1b = W1[_NUM_FIELDS * _EMB_DIM:].astype(bf16)
    return _mlp(xc3, x_num, w1p, w1b, b1.reshape(1, -1),
                W2.astype(bf16), b2.reshape(1, -1), W3.astype(bf16),
                b3.reshape(1, -1), W4.astype(bf16), b4.reshape(1, -1))


# speedup vs baseline: 3.8388x; 1.2042x over previous
import jax
import jax.numpy as jnp
from jax import lax
from jax.experimental import pallas as pl
from jax.experimental.pallas import tpu as pltpu
from jax.experimental.pallas import tpu_sc as plsc

_NUM_FIELDS = 26
_VOCAB = 1000
_EMB_DIM = 50
_EMB_PAD = 128
_HALF = 64
_NPAIR = _NUM_FIELDS // 2
_BATCH = 4096
_NW = 32
_BB = 128
_NBLK = _BATCH // _BB
_UNITS_PER_W = _NPAIR * _NBLK // _NW
_G = 128


def _sc_gather_body(xcat_hbm, table_hbm, out_hbm, xcv, idxb, rows_v,
                    sem, sem2):
    wid = lax.axis_index("s") * 2 + lax.axis_index("c")
    boff = wid * _BB
    pltpu.sync_copy(xcat_hbm.at[pl.ds(boff * _NUM_FIELDS,
                                      _BB * _NUM_FIELDS)], xcv)
    lanes = lax.iota(jnp.int32, 16)
    for f in range(_NUM_FIELDS):
        for k in range(_BB // 16):
            src = (k * 16 + lanes) * _NUM_FIELDS + f
            v = plsc.load_gather(xcv, [src]) + f * _VOCAB
            idxb[pl.ds(f * _BB + k * 16, 16)] = v
    store_cps = [None, None, None, None]
    for p in range(_NPAIR):
        b = p % 2
        for h in range(2):
            if store_cps[2 * b + h] is not None:
                store_cps[2 * b + h].wait()
        gcps = []
        for h in range(2):
            gcps.append(pltpu.async_copy(
                table_hbm.at[idxb.at[pl.ds((2 * p + h) * _BB, _BB)]],
                rows_v.at[b, h],
                sem,
            ))
        for cp in gcps:
            cp.wait()
        for h in range(2):
            store_cps[2 * b + h] = pltpu.async_copy(
                rows_v.at[b, h, slice(None), pl.ds(0, _HALF)],
                out_hbm.at[p, pl.ds(boff, _BB), pl.ds(h * _HALF, _HALF)],
                sem2,
            )
    for cp in store_cps:
        if cp is not None:
            cp.wait()


def _sc_gather(xcat_flat, table_pad):
    mesh = plsc.VectorSubcoreMesh(core_axis_name="c", subcore_axis_name="s")
    k = pl.kernel(
        _sc_gather_body,
        out_type=jax.ShapeDtypeStruct((_NPAIR, _BATCH, _EMB_PAD),
                                      jnp.float32),
        mesh=mesh,
        scratch_types=[
            pltpu.VMEM((_BB * _NUM_FIELDS,), jnp.int32),
            pltpu.VMEM((_BB * _NUM_FIELDS,), jnp.int32),
            pltpu.VMEM((2, 2, _BB, _EMB_PAD), jnp.float32),
            pltpu.SemaphoreType.DMA,
            pltpu.SemaphoreType.DMA,
        ],
        compiler_params=pltpu.CompilerParams(use_tc_tiling_on_sc=False,
                                             needs_layout_passes=False),
    )
    return k(xcat_flat, table_pad)


def _mlp_body(xc_ref, xn_ref, w1p_ref, w1b_ref, b1_ref, w2_ref, b2_ref,
              w3_ref, b3_ref, w4_ref, b4_ref, out_ref):
    f32, bf16 = jnp.float32, jnp.bfloat16
    h = jnp.dot(xn_ref[...].astype(bf16), w1b_ref[...],
                preferred_element_type=f32)
    x = jnp.concatenate([xc_ref[f] for f in range(_NPAIR)], axis=1)
    h += jnp.dot(x.astype(bf16), w1p_ref[...], preferred_element_type=f32)
    h = jnp.maximum(h + b1_ref[...], 0.0)
    e = jnp.dot(h.astype(bf16), w2_ref[...],
                preferred_element_type=f32) + b2_ref[...]
    h2 = jnp.dot(e.astype(bf16), w3_ref[...],
                 preferred_element_type=f32) + b3_ref[...]
    h2 = jnp.maximum(h2, 0.0)
    out_ref[...] = (jnp.dot(h2.astype(bf16), w4_ref[...],
                            preferred_element_type=f32) + b4_ref[...])


def _mlp(xc3, x_num, w1p, w1b, b1, w2, b2, w3, b3, w4, b4, bm=512):
    nb = _BATCH // bm
    full2 = lambda shape: pl.BlockSpec(shape, lambda i: (0, 0))
    return pl.pallas_call(
        _mlp_body,
        grid=(nb,),
        in_specs=[
            pl.BlockSpec((_NPAIR, bm, _EMB_PAD), lambda i: (0, i, 0)),
            pl.BlockSpec((bm, x_num.shape[1]), lambda i: (i, 0)),
            pl.BlockSpec(w1p.shape, lambda i: (0, 0)),
            full2(w1b.shape), full2(b1.shape),
            full2(w2.shape), full2(b2.shape),
            full2(w3.shape), full2(b3.shape),
            full2(w4.shape), full2(b4.shape),
        ],
        out_specs=pl.BlockSpec((bm, w4.shape[1]), lambda i: (i, 0)),
        out_shape=jax.ShapeDtypeStruct((_BATCH, w4.shape[1]), jnp.float32),
    )(xc3, x_num, w1p, w1b, b1, w2, b2, w3, b3, w4, b4)


@jax.jit
def kernel(x_cat, x_num, emb, W1, b1, W2, b2, W3, b3, W4, b4):
    table_pad = jnp.pad(
        emb.reshape(_NUM_FIELDS * _VOCAB, _EMB_DIM),
        ((0, 0), (0, _EMB_PAD - _EMB_DIM)))
    xc3 = _sc_gather(x_cat.reshape(-1), table_pad)
    bf16 = jnp.bfloat16
    w1r = (W1[:_NUM_FIELDS * _EMB_DIM].astype(bf16)
           .reshape(_NUM_FIELDS, _EMB_DIM, 1024))
    w1p = (jnp.zeros((_NPAIR, _EMB_PAD, 1024), bf16)
           .at[:, :_EMB_DIM, :].set(w1r[0::2])
           .at[:, _HALF:_HALF + _EMB_DIM, :].set(w1r[1::2])
           .reshape(_NPAIR * _EMB_PAD, 1024))
    w1b = W1[_NUM_FIELDS * _EMB_DIM:].astype(bf16)
    return _mlp(xc3, x_num, w1p, w1b, b1.reshape(1, -1),
                W2.astype(bf16), b2.reshape(1, -1), W3.astype(bf16),
                b3.reshape(1, -1), W4.astype(bf16), b4.reshape(1, -1))


# speedup vs baseline: 4.1608x; 1.0839x over previous
import jax
import jax.numpy as jnp
from jax import lax
from jax.experimental import pallas as pl
from jax.experimental.pallas import tpu as pltpu
from jax.experimental.pallas import tpu_sc as plsc

_NUM_FIELDS = 26
_VOCAB = 1000
_EMB_DIM = 50
_EMB_PAD = 128
_HALF = 64
_NPAIR = _NUM_FIELDS // 2
_BATCH = 4096
_NW = 32
_BB = 128
_NBLK = _BATCH // _BB
_UNITS_PER_W = _NPAIR * _NBLK // _NW
_G = 128


def _sc_gather_body(xcat_hbm, table_hbm, out_hbm, xcv, idxb, rows_v,
                    sem, sem2):
    wid = lax.axis_index("s") * 2 + lax.axis_index("c")
    boff = wid * _BB
    pltpu.sync_copy(xcat_hbm.at[pl.ds(boff * _NUM_FIELDS,
                                      _BB * _NUM_FIELDS)], xcv)
    lanes = lax.iota(jnp.int32, 16)
    for f in range(_NUM_FIELDS):
        for k in range(_BB // 16):
            src = (k * 16 + lanes) * _NUM_FIELDS + f
            v = plsc.load_gather(xcv, [src]) + f * _VOCAB
            idxb[pl.ds(f * _BB + k * 16, 16)] = v
    def fire_gathers(p, b):
        return [pltpu.async_copy(
            table_hbm.at[idxb.at[pl.ds((2 * p + h) * _BB, _BB)]],
            rows_v.at[b, h], sem) for h in range(2)]

    def fire_stores(p, b):
        return [pltpu.async_copy(
            rows_v.at[b, h, slice(None), pl.ds(0, _HALF)],
            out_hbm.at[p, pl.ds(boff, _BB), pl.ds(h * _HALF, _HALF)],
            sem2) for h in range(2)]

    store_cps = [None, None]
    prev_g = fire_gathers(0, 0)
    for p in range(1, _NPAIR + 1):
        b = p % 2
        if p < _NPAIR:
            if store_cps[b] is not None:
                for cp in store_cps[b]:
                    cp.wait()
            gcur = fire_gathers(p, b)
        for cp in prev_g:
            cp.wait()
        store_cps[1 - b] = fire_stores(p - 1, 1 - b)
        if p < _NPAIR:
            prev_g = gcur
    for cps in store_cps:
        if cps is not None:
            for cp in cps:
                cp.wait()


def _sc_gather(xcat_flat, table_pad):
    mesh = plsc.VectorSubcoreMesh(core_axis_name="c", subcore_axis_name="s")
    k = pl.kernel(
        _sc_gather_body,
        out_type=jax.ShapeDtypeStruct((_NPAIR, _BATCH, _EMB_PAD),
                                      jnp.float32),
        mesh=mesh,
        scratch_types=[
            pltpu.VMEM((_BB * _NUM_FIELDS,), jnp.int32),
            pltpu.VMEM((_BB * _NUM_FIELDS,), jnp.int32),
            pltpu.VMEM((2, 2, _BB, _EMB_PAD), jnp.float32),
            pltpu.SemaphoreType.DMA,
            pltpu.SemaphoreType.DMA,
        ],
        compiler_params=pltpu.CompilerParams(use_tc_tiling_on_sc=False,
                                             needs_layout_passes=False),
    )
    return k(xcat_flat, table_pad)


def _mlp_body(xc_ref, xn_ref, w1p_ref, w1b_ref, b1_ref, w2_ref, b2_ref,
              w3_ref, b3_ref, w4_ref, b4_ref, out_ref):
    f32, bf16 = jnp.float32, jnp.bfloat16
    h = jnp.dot(xn_ref[...].astype(bf16), w1b_ref[...],
                preferred_element_type=f32)
    x = jnp.concatenate([xc_ref[f] for f in range(_NPAIR)], axis=1)
    h += jnp.dot(x.astype(bf16), w1p_ref[...], preferred_element_type=f32)
    h = jnp.maximum(h + b1_ref[...], 0.0)
    e = jnp.dot(h.astype(bf16), w2_ref[...],
                preferred_element_type=f32) + b2_ref[...]
    h2 = jnp.dot(e.astype(bf16), w3_ref[...],
                 preferred_element_type=f32) + b3_ref[...]
    h2 = jnp.maximum(h2, 0.0)
    out_ref[...] = (jnp.dot(h2.astype(bf16), w4_ref[...],
                            preferred_element_type=f32) + b4_ref[...])


def _mlp(xc3, x_num, w1p, w1b, b1, w2, b2, w3, b3, w4, b4, bm=512):
    nb = _BATCH // bm
    full2 = lambda shape: pl.BlockSpec(shape, lambda i: (0, 0))
    return pl.pallas_call(
        _mlp_body,
        grid=(nb,),
        in_specs=[
            pl.BlockSpec((_NPAIR, bm, _EMB_PAD), lambda i: (0, i, 0)),
            pl.BlockSpec((bm, x_num.shape[1]), lambda i: (i, 0)),
            pl.BlockSpec(w1p.shape, lambda i: (0, 0)),
            full2(w1b.shape), full2(b1.shape),
            full2(w2.shape), full2(b2.shape),
            full2(w3.shape), full2(b3.shape),
            full2(w4.shape), full2(b4.shape),
        ],
        out_specs=pl.BlockSpec((bm, w4.shape[1]), lambda i: (i, 0)),
        out_shape=jax.ShapeDtypeStruct((_BATCH, w4.shape[1]), jnp.float32),
    )(xc3, x_num, w1p, w1b, b1, w2, b2, w3, b3, w4, b4)


@jax.jit
def kernel(x_cat, x_num, emb, W1, b1, W2, b2, W3, b3, W4, b4):
    table_pad = jnp.pad(
        emb, ((0, 0), (0, 0), (0, _EMB_PAD - _EMB_DIM))
    ).reshape(_NUM_FIELDS * _VOCAB, _EMB_PAD)
    xc3 = _sc_gather(x_cat.reshape(-1), table_pad)
    bf16 = jnp.bfloat16
    w1r = (W1[:_NUM_FIELDS * _EMB_DIM].astype(bf16)
           .reshape(_NUM_FIELDS, _EMB_DIM, 1024))
    w1p = jnp.pad(w1r, ((0, 0), (0, _HALF - _EMB_DIM), (0, 0))
                  ).reshape(_NPAIR * _EMB_PAD, 1024)
    w1b = W1[_NUM_FIELDS * _EMB_DIM:].astype(bf16)
    return _mlp(xc3, x_num, w1p, w1b, b1.reshape(1, -1),
                W2.astype(bf16), b2.reshape(1, -1), W3.astype(bf16),
                b3.reshape(1, -1), W4.astype(bf16), b4.reshape(1, -1))
